# baseline (speedup 1.0000x reference)
import functools

import jax
import jax.numpy as jnp
from jax import lax
from jax.experimental import pallas as pl
from jax.experimental.pallas import tpu as pltpu
from jax.experimental.pallas import tpu_sc as plsc

_NUM_POS = 8192
_EMBED_DIM = 768
_B = 4
_S = 8192
_TOTAL = _B * _S

_NC = 2
_NS = 16
_NW = _NC * _NS
_ROWS_PER_W = _TOTAL // _NW
_CHUNK = 32
_N_CHUNKS = _ROWS_PER_W // _CHUNK
_NBUF = 4

_mesh = plsc.VectorSubcoreMesh(core_axis_name="c", subcore_axis_name="s")


@functools.partial(
    pl.kernel,
    mesh=_mesh,
    out_type=jax.ShapeDtypeStruct((_TOTAL, _EMBED_DIM), jnp.float32),
    scratch_types=[
        pltpu.VMEM((_ROWS_PER_W,), jnp.int32),
        pltpu.VMEM((_NBUF, _CHUNK, _EMBED_DIM), jnp.float32),
        pltpu.SemaphoreType.DMA,
        pltpu.SemaphoreType.DMA,
    ],
)
def _gather_rows(idx_hbm, table_hbm, out_hbm, idx_v, rows_v, sg, sw):
    sid = lax.axis_index("s")
    wid = sid * _NC + lax.axis_index("c")
    base = wid * _ROWS_PER_W

    pltpu.sync_copy(
        idx_hbm.at[wid // 8, pl.ds((wid % 8) * _ROWS_PER_W, _ROWS_PER_W)],
        idx_v)

    def _gather(i, b):
        pltpu.async_copy(
            table_hbm.at[idx_v.at[pl.ds(i * _CHUNK, _CHUNK)]],
            rows_v.at[b], sg)

    def _wait_gather():
        pltpu.make_async_copy(
            table_hbm.at[idx_v.at[pl.ds(0, _CHUNK)]], rows_v.at[0],
            sg).wait()

    def _write_out(i, b):
        pltpu.async_copy(rows_v.at[b],
                         out_hbm.at[pl.ds(base + i * _CHUNK, _CHUNK)], sw)

    def _wait_write():
        pltpu.make_async_copy(
            rows_v.at[0], out_hbm.at[pl.ds(base, _CHUNK)], sw).wait()

    for b in range(_NBUF):
        _gather(b, b)

    def _body(i, carry):
        b = lax.rem(i, _NBUF)
        _wait_gather()
        _write_out(i, b)
        _wait_write()
        _gather(i + _NBUF, b)
        return carry

    lax.fori_loop(0, _N_CHUNKS - _NBUF, _body, 0)

    def _tail(i, carry):
        _wait_gather()
        _write_out(i, lax.rem(i, _NBUF))
        _wait_write()
        return carry

    lax.fori_loop(_N_CHUNKS - _NBUF, _N_CHUNKS, _tail, 0)


def _copy_body(w_ref, o_ref):
    o_ref[...] = w_ref[...]


def _weights_passthrough(w):
    return pl.pallas_call(
        _copy_body,
        grid=(2,),
        in_specs=[pl.BlockSpec((_NUM_POS // 2, _EMBED_DIM),
                               lambda i: (i, 0))],
        out_specs=pl.BlockSpec((_NUM_POS // 2, _EMBED_DIM),
                               lambda i: (i, 0)),
        out_shape=jax.ShapeDtypeStruct((_NUM_POS, _EMBED_DIM), jnp.float32),
    )(w)


def kernel(inputs, pos_embed_weights):
    idx = inputs.astype(jnp.int32)
    out = _gather_rows(idx, pos_embed_weights)
    w_out = _weights_passthrough(pos_embed_weights)
    return out.reshape(_B, _S, _EMBED_DIM), w_out


# speedup vs baseline: 1.0019x; 1.0019x over previous
import functools

import jax
import jax.numpy as jnp
from jax import lax
from jax.experimental import pallas as pl
from jax.experimental.pallas import tpu as pltpu
from jax.experimental.pallas import tpu_sc as plsc

_NUM_POS = 8192
_EMBED_DIM = 768
_B = 4
_S = 8192
_TOTAL = _B * _S

_NC = 2
_NS = 16
_NW = _NC * _NS
_ROWS_PER_W = _TOTAL // _NW
_CHUNK = 32
_N_CHUNKS = _ROWS_PER_W // _CHUNK
_NBUF = 4

_mesh = plsc.VectorSubcoreMesh(core_axis_name="c", subcore_axis_name="s")


@functools.partial(
    pl.kernel,
    mesh=_mesh,
    out_type=jax.ShapeDtypeStruct((_TOTAL, _EMBED_DIM), jnp.float32),
    scratch_types=[
        pltpu.VMEM((_ROWS_PER_W,), jnp.int32),
        pltpu.VMEM((_NBUF, _CHUNK, _EMBED_DIM), jnp.float32),
        pltpu.SemaphoreType.DMA,
        pltpu.SemaphoreType.DMA,
    ],
)
def _gather_rows(idx_hbm, table_hbm, out_hbm, idx_v, rows_v, sg, sw):
    sid = lax.axis_index("s")
    wid = sid * _NC + lax.axis_index("c")
    base = wid * _ROWS_PER_W

    pltpu.sync_copy(
        idx_hbm.at[wid // 8, pl.ds((wid % 8) * _ROWS_PER_W, _ROWS_PER_W)],
        idx_v)

    def _gather(i, b):
        pltpu.async_copy(
            table_hbm.at[idx_v.at[pl.ds(i * _CHUNK, _CHUNK)]],
            rows_v.at[b], sg)

    def _wait_gather():
        pltpu.make_async_copy(
            table_hbm.at[idx_v.at[pl.ds(0, _CHUNK)]], rows_v.at[0],
            sg).wait()

    def _write_out(i, b):
        pltpu.async_copy(rows_v.at[b],
                         out_hbm.at[pl.ds(base + i * _CHUNK, _CHUNK)], sw)

    def _wait_write():
        pltpu.make_async_copy(
            rows_v.at[0], out_hbm.at[pl.ds(base, _CHUNK)], sw).wait()

    for b in range(_NBUF):
        _gather(b, b)

    def _body(i, carry):
        b = lax.rem(i, _NBUF)
        _wait_gather()
        _write_out(i, b)
        _wait_write()
        _gather(i + _NBUF, b)
        return carry

    lax.fori_loop(0, _N_CHUNKS - _NBUF, _body, 0)

    def _tail(i, carry):
        _wait_gather()
        _write_out(i, lax.rem(i, _NBUF))
        _wait_write()
        return carry

    lax.fori_loop(_N_CHUNKS - _NBUF, _N_CHUNKS, _tail, 0)


def _copy_body(w_ref, o_ref):
    o_ref[...] = w_ref[...]


def _weights_passthrough(w):
    return pl.pallas_call(
        _copy_body,
        grid=(4,),
        in_specs=[pl.BlockSpec((_NUM_POS // 4, _EMBED_DIM),
                               lambda i: (i, 0))],
        out_specs=pl.BlockSpec((_NUM_POS // 4, _EMBED_DIM),
                               lambda i: (i, 0)),
        out_shape=jax.ShapeDtypeStruct((_NUM_POS, _EMBED_DIM), jnp.float32),
    )(w)


def kernel(inputs, pos_embed_weights):
    idx = inputs.astype(jnp.int32)
    out = _gather_rows(idx, pos_embed_weights)
    w_out = _weights_passthrough(pos_embed_weights)
    return out.reshape(_B, _S, _EMBED_DIM), w_out


# speedup vs baseline: 1.0073x; 1.0054x over previous
import functools

import jax
import jax.numpy as jnp
from jax import lax
from jax.experimental import pallas as pl
from jax.experimental.pallas import tpu as pltpu
from jax.experimental.pallas import tpu_sc as plsc

_NUM_POS = 8192
_EMBED_DIM = 768
_B = 4
_S = 8192
_TOTAL = _B * _S

_NC = 2
_NS = 16
_NW = _NC * _NS
_ROWS_PER_W = _TOTAL // _NW
_CHUNK = 32
_N_CHUNKS = _ROWS_PER_W // _CHUNK
_NBUF = 5

_mesh = plsc.VectorSubcoreMesh(core_axis_name="c", subcore_axis_name="s")


@functools.partial(
    pl.kernel,
    mesh=_mesh,
    out_type=jax.ShapeDtypeStruct((_TOTAL, _EMBED_DIM), jnp.float32),
    scratch_types=[
        pltpu.VMEM((_ROWS_PER_W,), jnp.int32),
        pltpu.VMEM((_NBUF, _CHUNK, _EMBED_DIM), jnp.float32),
        pltpu.SemaphoreType.DMA,
        pltpu.SemaphoreType.DMA,
    ],
)
def _gather_rows(idx_hbm, table_hbm, out_hbm, idx_v, rows_v, sg, sw):
    sid = lax.axis_index("s")
    wid = sid * _NC + lax.axis_index("c")
    base = wid * _ROWS_PER_W

    pltpu.sync_copy(
        idx_hbm.at[wid // 8, pl.ds((wid % 8) * _ROWS_PER_W, _ROWS_PER_W)],
        idx_v)

    def _gather(i, b):
        pltpu.async_copy(
            table_hbm.at[idx_v.at[pl.ds(i * _CHUNK, _CHUNK)]],
            rows_v.at[b], sg)

    def _wait_gather():
        pltpu.make_async_copy(
            table_hbm.at[idx_v.at[pl.ds(0, _CHUNK)]], rows_v.at[0],
            sg).wait()

    def _write_out(i, b):
        pltpu.async_copy(rows_v.at[b],
                         out_hbm.at[pl.ds(base + i * _CHUNK, _CHUNK)], sw)

    def _wait_write():
        pltpu.make_async_copy(
            rows_v.at[0], out_hbm.at[pl.ds(base, _CHUNK)], sw).wait()

    for b in range(_NBUF):
        _gather(b, b)

    def _body(i, carry):
        b = lax.rem(i, _NBUF)
        _wait_gather()
        _write_out(i, b)
        _wait_write()
        _gather(i + _NBUF, b)
        return carry

    lax.fori_loop(0, _N_CHUNKS - _NBUF, _body, 0)

    def _tail(i, carry):
        _wait_gather()
        _write_out(i, lax.rem(i, _NBUF))
        _wait_write()
        return carry

    lax.fori_loop(_N_CHUNKS - _NBUF, _N_CHUNKS, _tail, 0)


def _copy_body(w_ref, o_ref):
    o_ref[...] = w_ref[...]


def _weights_passthrough(w):
    return pl.pallas_call(
        _copy_body,
        grid=(4,),
        in_specs=[pl.BlockSpec((_NUM_POS // 4, _EMBED_DIM),
                               lambda i: (i, 0))],
        out_specs=pl.BlockSpec((_NUM_POS // 4, _EMBED_DIM),
                               lambda i: (i, 0)),
        out_shape=jax.ShapeDtypeStruct((_NUM_POS, _EMBED_DIM), jnp.float32),
    )(w)


def kernel(inputs, pos_embed_weights):
    idx = inputs.astype(jnp.int32)
    out = _gather_rows(idx, pos_embed_weights)
    w_out = _weights_passthrough(pos_embed_weights)
    return out.reshape(_B, _S, _EMBED_DIM), w_out
